# baseline (speedup 1.0000x reference)
import jax
import jax.numpy as jnp
from jax.experimental import pallas as pl
from jax.experimental.pallas import tpu as pltpu

_SHAPE = (2, 16384, 1024)
_FLAT_ROWS = 2 * _SHAPE[1]
_CHUNK_ROWS = (
    [128, 128, 256, 512, 1024, 2048] + [3200] * 8 + [1536, 1024, 512]
)
_OFFSETS = [sum(_CHUNK_ROWS[:i]) for i in range(len(_CHUNK_ROWS))]
assert sum(_CHUNK_ROWS) == _FLAT_ROWS
_N_CHUNKS = len(_CHUNK_ROWS)
_MAX_ROWS = max(_CHUNK_ROWS)
_NBUF = 4
_PATCH_ROW = _SHAPE[1]
_PATCH_CHUNK = next(
    i
    for i, (o, r) in enumerate(zip(_OFFSETS, _CHUNK_ROWS))
    if o <= _PATCH_ROW and _PATCH_ROW + 8 <= o + r
)


def _copy_update_kernel(key_ref, in_ref, out_ref, bufs, sem_in, sem_out):
    def start_in(i):
        pltpu.make_async_copy(
            in_ref.at[pl.ds(_OFFSETS[i], _CHUNK_ROWS[i]), :],
            bufs.at[i % _NBUF, pl.ds(0, _CHUNK_ROWS[i]), :],
            sem_in.at[i % _NBUF],
        ).start()

    def wait_in(i):
        pltpu.make_async_copy(
            in_ref.at[pl.ds(_OFFSETS[i], _CHUNK_ROWS[i]), :],
            bufs.at[i % _NBUF, pl.ds(0, _CHUNK_ROWS[i]), :],
            sem_in.at[i % _NBUF],
        ).wait()

    def start_out(i):
        pltpu.make_async_copy(
            bufs.at[i % _NBUF, pl.ds(0, _CHUNK_ROWS[i]), :],
            out_ref.at[pl.ds(_OFFSETS[i], _CHUNK_ROWS[i]), :],
            sem_out.at[i % _NBUF],
        ).start()

    def wait_out(i):
        pltpu.make_async_copy(
            bufs.at[i % _NBUF, pl.ds(0, _CHUNK_ROWS[i]), :],
            out_ref.at[pl.ds(_OFFSETS[i], _CHUNK_ROWS[i]), :],
            sem_out.at[i % _NBUF],
        ).wait()

    lookahead = _NBUF // 2
    for i in range(lookahead):
        start_in(i)
    for i in range(_N_CHUNKS):
        nxt = i + lookahead
        if nxt < _N_CHUNKS:
            if nxt >= _NBUF:
                wait_out(nxt - _NBUF)
            start_in(nxt)
        wait_in(i)
        if i == _PATCH_CHUNK:
            base = _PATCH_ROW - _OFFSETS[i]
            row = jax.lax.broadcasted_iota(jnp.int32, (8, 128), 0)
            col = jax.lax.broadcasted_iota(jnp.int32, (8, 128), 1)
            mask = (row == 0) & (col == 1)
            bufs[i % _NBUF, base : base + 8, 0:128] += jnp.where(
                mask, 2.0 * key_ref[0], 0.0
            )
        start_out(i)
    for i in range(max(0, _N_CHUNKS - _NBUF), _N_CHUNKS):
        wait_out(i)


def kernel(key, cache_next):
    flat = cache_next.reshape(_FLAT_ROWS, _SHAPE[2])
    out = pl.pallas_call(
        _copy_update_kernel,
        out_shape=jax.ShapeDtypeStruct((_FLAT_ROWS, _SHAPE[2]), jnp.float32),
        in_specs=[
            pl.BlockSpec(memory_space=pltpu.SMEM),
            pl.BlockSpec(memory_space=pl.ANY),
        ],
        out_specs=pl.BlockSpec(memory_space=pl.ANY),
        scratch_shapes=[
            pltpu.VMEM((_NBUF, _MAX_ROWS, _SHAPE[2]), jnp.float32),
            pltpu.SemaphoreType.DMA((_NBUF,)),
            pltpu.SemaphoreType.DMA((_NBUF,)),
        ],
    )(key, flat)
    return key, out.reshape(_SHAPE)


# speedup vs baseline: 1.0076x; 1.0076x over previous
import jax
import jax.numpy as jnp
from jax.experimental import pallas as pl
from jax.experimental.pallas import tpu as pltpu

_SHAPE = (2, 16384, 1024)
_FLAT_ROWS = 2 * _SHAPE[1]
_CHUNK_ROWS = [512, 1024, 2048] + [3392] * 8 + [2048]
_OFFSETS = [sum(_CHUNK_ROWS[:i]) for i in range(len(_CHUNK_ROWS))]
assert sum(_CHUNK_ROWS) == _FLAT_ROWS
_N_CHUNKS = len(_CHUNK_ROWS)
_MAX_ROWS = max(_CHUNK_ROWS)
_NBUF = 4
_PATCH_ROW = _SHAPE[1]
_PATCH_CHUNK = next(
    i
    for i, (o, r) in enumerate(zip(_OFFSETS, _CHUNK_ROWS))
    if o <= _PATCH_ROW and _PATCH_ROW + 8 <= o + r
)


def _copy_update_kernel(key_ref, in_ref, out_ref, bufs, sem_in, sem_out):
    def start_in(i):
        pltpu.make_async_copy(
            in_ref.at[pl.ds(_OFFSETS[i], _CHUNK_ROWS[i]), :],
            bufs.at[i % _NBUF, pl.ds(0, _CHUNK_ROWS[i]), :],
            sem_in.at[i % _NBUF],
        ).start()

    def wait_in(i):
        pltpu.make_async_copy(
            in_ref.at[pl.ds(_OFFSETS[i], _CHUNK_ROWS[i]), :],
            bufs.at[i % _NBUF, pl.ds(0, _CHUNK_ROWS[i]), :],
            sem_in.at[i % _NBUF],
        ).wait()

    def start_out(i):
        pltpu.make_async_copy(
            bufs.at[i % _NBUF, pl.ds(0, _CHUNK_ROWS[i]), :],
            out_ref.at[pl.ds(_OFFSETS[i], _CHUNK_ROWS[i]), :],
            sem_out.at[i % _NBUF],
        ).start()

    def wait_out(i):
        pltpu.make_async_copy(
            bufs.at[i % _NBUF, pl.ds(0, _CHUNK_ROWS[i]), :],
            out_ref.at[pl.ds(_OFFSETS[i], _CHUNK_ROWS[i]), :],
            sem_out.at[i % _NBUF],
        ).wait()

    lookahead = _NBUF // 2
    for i in range(lookahead):
        start_in(i)
    for i in range(_N_CHUNKS):
        nxt = i + lookahead
        if nxt < _N_CHUNKS:
            if nxt >= _NBUF:
                wait_out(nxt - _NBUF)
            start_in(nxt)
        wait_in(i)
        if i == _PATCH_CHUNK:
            base = _PATCH_ROW - _OFFSETS[i]
            row = jax.lax.broadcasted_iota(jnp.int32, (8, 128), 0)
            col = jax.lax.broadcasted_iota(jnp.int32, (8, 128), 1)
            mask = (row == 0) & (col == 1)
            bufs[i % _NBUF, base : base + 8, 0:128] += jnp.where(
                mask, 2.0 * key_ref[0], 0.0
            )
        start_out(i)
    for i in range(max(0, _N_CHUNKS - _NBUF), _N_CHUNKS):
        wait_out(i)


def kernel(key, cache_next):
    flat = cache_next.reshape(_FLAT_ROWS, _SHAPE[2])
    out = pl.pallas_call(
        _copy_update_kernel,
        out_shape=jax.ShapeDtypeStruct((_FLAT_ROWS, _SHAPE[2]), jnp.float32),
        in_specs=[
            pl.BlockSpec(memory_space=pltpu.SMEM),
            pl.BlockSpec(memory_space=pl.ANY),
        ],
        out_specs=pl.BlockSpec(memory_space=pl.ANY),
        scratch_shapes=[
            pltpu.VMEM((_NBUF, _MAX_ROWS, _SHAPE[2]), jnp.float32),
            pltpu.SemaphoreType.DMA((_NBUF,)),
            pltpu.SemaphoreType.DMA((_NBUF,)),
        ],
    )(key, flat)
    return key, out.reshape(_SHAPE)


# speedup vs baseline: 1.0086x; 1.0010x over previous
import jax
import jax.numpy as jnp
from jax.experimental import pallas as pl
from jax.experimental.pallas import tpu as pltpu

_SHAPE = (2, 16384, 1024)
_FLAT_ROWS = 2 * _SHAPE[1]
_CHUNK_ROWS = [512, 1024, 2048] + [3200] * 8 + [2048, 1024, 512]
_OFFSETS = [sum(_CHUNK_ROWS[:i]) for i in range(len(_CHUNK_ROWS))]
assert sum(_CHUNK_ROWS) == _FLAT_ROWS
_N_CHUNKS = len(_CHUNK_ROWS)
_MAX_ROWS = max(_CHUNK_ROWS)
_NBUF = 4
_PATCH_ROW = _SHAPE[1]
_PATCH_CHUNK = next(
    i
    for i, (o, r) in enumerate(zip(_OFFSETS, _CHUNK_ROWS))
    if o <= _PATCH_ROW and _PATCH_ROW + 8 <= o + r
)


def _copy_update_kernel(key_ref, in_ref, out_ref, bufs, sem_in, sem_out):
    def start_in(i):
        pltpu.make_async_copy(
            in_ref.at[pl.ds(_OFFSETS[i], _CHUNK_ROWS[i]), :],
            bufs.at[i % _NBUF, pl.ds(0, _CHUNK_ROWS[i]), :],
            sem_in.at[i % _NBUF],
        ).start()

    def wait_in(i):
        pltpu.make_async_copy(
            in_ref.at[pl.ds(_OFFSETS[i], _CHUNK_ROWS[i]), :],
            bufs.at[i % _NBUF, pl.ds(0, _CHUNK_ROWS[i]), :],
            sem_in.at[i % _NBUF],
        ).wait()

    def start_out(i):
        pltpu.make_async_copy(
            bufs.at[i % _NBUF, pl.ds(0, _CHUNK_ROWS[i]), :],
            out_ref.at[pl.ds(_OFFSETS[i], _CHUNK_ROWS[i]), :],
            sem_out.at[i % _NBUF],
        ).start()

    def wait_out(i):
        pltpu.make_async_copy(
            bufs.at[i % _NBUF, pl.ds(0, _CHUNK_ROWS[i]), :],
            out_ref.at[pl.ds(_OFFSETS[i], _CHUNK_ROWS[i]), :],
            sem_out.at[i % _NBUF],
        ).wait()

    lookahead = _NBUF // 2
    for i in range(lookahead):
        start_in(i)
    for i in range(_N_CHUNKS):
        nxt = i + lookahead
        if nxt < _N_CHUNKS:
            if nxt >= _NBUF:
                wait_out(nxt - _NBUF)
            start_in(nxt)
        wait_in(i)
        if i == _PATCH_CHUNK:
            base = _PATCH_ROW - _OFFSETS[i]
            row = jax.lax.broadcasted_iota(jnp.int32, (8, 128), 0)
            col = jax.lax.broadcasted_iota(jnp.int32, (8, 128), 1)
            mask = (row == 0) & (col == 1)
            bufs[i % _NBUF, base : base + 8, 0:128] += jnp.where(
                mask, 2.0 * key_ref[0], 0.0
            )
        start_out(i)
    for i in range(max(0, _N_CHUNKS - _NBUF), _N_CHUNKS):
        wait_out(i)


def kernel(key, cache_next):
    flat = cache_next.reshape(_FLAT_ROWS, _SHAPE[2])
    out = pl.pallas_call(
        _copy_update_kernel,
        out_shape=jax.ShapeDtypeStruct((_FLAT_ROWS, _SHAPE[2]), jnp.float32),
        in_specs=[
            pl.BlockSpec(memory_space=pltpu.SMEM),
            pl.BlockSpec(memory_space=pl.ANY),
        ],
        out_specs=pl.BlockSpec(memory_space=pl.ANY),
        scratch_shapes=[
            pltpu.VMEM((_NBUF, _MAX_ROWS, _SHAPE[2]), jnp.float32),
            pltpu.SemaphoreType.DMA((_NBUF,)),
            pltpu.SemaphoreType.DMA((_NBUF,)),
        ],
    )(key, flat)
    return key, out.reshape(_SHAPE)
